# baseline (speedup 1.0000x reference)
import functools

import numpy as np
import jax
import jax.numpy as jnp
from jax import lax
from jax.experimental import pallas as pl
from jax.experimental.pallas import tpu as pltpu
from jax.experimental.pallas import tpu_sc as plsc

N = 10000
K = 16
H = 64
NP = 10240
QB = 400
GB = 200
LB = 1000
RB = 1000
E = N * K
E_PAD = 163840
NW = 32
PER_W = E_PAD // NW
SUB = 512

_SQRT_HALF = np.sqrt(0.5).astype(np.float32)
_BIG = float(np.inf)

_P9 = [2.326819970068386e-2, -1.387039388740657e-1, 3.687424674597105e-1,
       -5.824733027278666e-1, 6.210004621745983e-1, -4.944515323274145e-1,
       3.404879937665872e-1, -2.741127028184656e-1, 5.638259427386472e-1]
_R8 = [-1.047766399936249e+1, 1.297719955372516e+1, -7.495518717768503e+0,
       2.921019019210786e+0, -1.015265279202700e+0, 4.218463358204948e-1,
       -2.820767439740514e-1, 5.641895067754075e-1]
_T7 = [7.853861353153693e-5, -8.010193625184903e-4, 5.188327685732524e-3,
       -2.685381193529856e-2, 1.128358514861418e-1, -3.761262582423300e-1,
       1.128379165726710e+0]


def _poly(y, coeffs):
    p = jnp.zeros_like(y)
    for c in coeffs:
        p = p * y + np.float32(c)
    return p


def _erfc(w):
    ax = jnp.abs(w)
    xsq = w * w
    z = jnp.exp(-xsq)
    q = 1.0 / ax
    y = 1.0 / xsq
    p = jnp.where(ax < 2.0, _poly(y, _P9), _poly(y, _R8))
    yv = (z * q) * p
    yc = jnp.where(w < 0.0, 2.0 - yv, yv)
    sm = 1.0 - w * _poly(xsq, _T7)
    return jnp.where(ax < 1.0, sm, yc)


def _gelu_exact(x):
    return 0.5 * x * _erfc((-x) * _SQRT_HALF)


def _lane_sum64(a):
    acc = a[:, 0:8]
    for t in range(1, 8):
        acc = acc + a[:, 8 * t : 8 * t + 8]
    acc = acc[:, 0:4] + acc[:, 4:8]
    acc = acc[:, 0:2] + acc[:, 2:4]
    return acc[:, 0:1] + acc[:, 1:2]


def _mean_axis1_16(t3):
    acc = t3[:, 0, :]
    for j in range(1, K):
        acc = acc + t3[:, j, :]
    return acc / 16.0


def _lift_body(inp_ref, w_ref, b_ref, out_ref):
    out_ref[...] = (
        jnp.dot(inp_ref[...], w_ref[...], preferred_element_type=jnp.float32)
        + b_ref[...]
    )


def _lift(inp, w, b):
    return pl.pallas_call(
        _lift_body,
        grid=(N // LB,),
        in_specs=[
            pl.BlockSpec((LB, inp.shape[1]), lambda i: (i, 0)),
            pl.BlockSpec(w.shape, lambda i: (0, 0)),
            pl.BlockSpec((1, w.shape[1]), lambda i: (0, 0)),
        ],
        out_specs=pl.BlockSpec((LB, w.shape[1]), lambda i: (i, 0)),
        out_shape=jax.ShapeDtypeStruct((N, w.shape[1]), jnp.float32),
    )(inp, w, b.reshape(1, -1))


def _knn_body(cq_ref, ct_ref, idx_ref):
    qx = cq_ref[:, 0:1]
    qy = cq_ref[:, 1:2]
    qz = cq_ref[:, 2:3]
    cx = ct_ref[0:1, :]
    cy = ct_ref[1:2, :]
    cz = ct_ref[2:3, :]
    dx = qx - cx
    dy = qy - cy
    dz = qz - cz
    d = (dx * dx + dy * dy) + dz * dz
    lane = lax.broadcasted_iota(jnp.int32, (QB, NP), 1)
    for k in range(K):
        pos = jnp.argmin(d, axis=1).astype(jnp.int32)
        idx_ref[:, k : k + 1] = pos[:, None]
        d = jnp.where(lane == pos[:, None], _BIG, d)


def _knn(coord_q, coord_t):
    return pl.pallas_call(
        _knn_body,
        grid=(N // QB,),
        in_specs=[
            pl.BlockSpec((QB, 8), lambda i: (i, 0)),
            pl.BlockSpec((8, NP), lambda i: (0, 0)),
        ],
        out_specs=pl.BlockSpec((QB, K), lambda i: (i, 0)),
        out_shape=jax.ShapeDtypeStruct((N, K), jnp.int32),
    )(coord_q, coord_t)


def _sc_gather(idx_pad, table):
    d = table.shape[1]
    mesh = plsc.VectorSubcoreMesh(core_axis_name="c", subcore_axis_name="s")

    n_chunks = PER_W // SUB

    @functools.partial(
        pl.kernel,
        out_type=jax.ShapeDtypeStruct((E_PAD, d), jnp.float32),
        mesh=mesh,
        compiler_params=pltpu.CompilerParams(use_tc_tiling_on_sc=False),
        scratch_types=[
            pltpu.VMEM((PER_W,), jnp.int32),
            pltpu.VMEM((2, SUB, d), jnp.float32),
            pltpu.SemaphoreType.DMA,
            pltpu.SemaphoreType.DMA,
        ],
    )
    def gk(idx_hbm, table_hbm, out_hbm, idx_v, rows_v, gsem, ssem):
        wid = lax.axis_index("s") * 2 + lax.axis_index("c")
        base_w = wid * PER_W
        pltpu.sync_copy(idx_hbm.at[pl.ds(base_w, PER_W)], idx_v)

        def fire(t, b):
            hs = []
            for q in range(SUB // 128):
                hs.append(
                    pltpu.async_copy(
                        table_hbm.at[idx_v.at[pl.ds(t * SUB + q * 128, 128)]],
                        rows_v.at[b].at[pl.ds(q * 128, 128)],
                        gsem,
                    )
                )
            return hs

        gh = {0: fire(0, 0)}
        sh = {}
        for t in range(n_chunks):
            b = t % 2
            if t + 1 < n_chunks:
                if t >= 1:
                    sh.pop(t - 1).wait()
                gh[t + 1] = fire(t + 1, 1 - b)
            for hnd in gh.pop(t):
                hnd.wait()
            sh[t] = pltpu.async_copy(
                rows_v.at[b], out_hbm.at[pl.ds(base_w + t * SUB, SUB)], ssem
            )
        sh.pop(n_chunks - 2).wait()
        sh.pop(n_chunks - 1).wait()

    return gk(idx_pad, table)


def _gno_body(nbrh_ref, nbrc_ref, ce_ref, hq_ref, k1w_ref, k1b_ref,
              k2w_ref, k2b_ref, sw_ref, sb_ref, ng_ref, nb_ref, out_ref):
    rel = nbrc_ref[:, 0:3] - ce_ref[:, 0:3]
    ki = jnp.concatenate([rel, nbrh_ref[...]], axis=1)
    t1 = (
        jnp.dot(ki, k1w_ref[...], preferred_element_type=jnp.float32)
        + k1b_ref[...]
    )
    g = _gelu_exact(t1)
    t2 = (
        jnp.dot(g, k2w_ref[...], preferred_element_type=jnp.float32)
        + k2b_ref[...]
    )
    agg = _mean_axis1_16(t2.reshape(GB, K, H))
    y = jnp.maximum(
        (agg + jnp.dot(hq_ref[...], sw_ref[...],
                       preferred_element_type=jnp.float32)) + sb_ref[...],
        0,
    )
    m = _lane_sum64(y) / 64.0
    c = y - m
    v = _lane_sum64(c * c) / 64.0
    out_ref[...] = (y - m) / jnp.sqrt(v + 1e-5) * ng_ref[...] + nb_ref[...]


def _gno(nbrh, nbrc, coord_e, hq, k1w, k1b, k2w, k2b, sw, sb, ng, nb):
    row = lambda a: a.reshape(1, -1)
    full = lambda a: pl.BlockSpec(a.shape, lambda i: (0, 0))
    return pl.pallas_call(
        _gno_body,
        grid=(N // GB,),
        in_specs=[
            pl.BlockSpec((GB * K, H), lambda i: (i, 0)),
            pl.BlockSpec((GB * K, 16), lambda i: (i, 0)),
            pl.BlockSpec((GB * K, 8), lambda i: (i, 0)),
            pl.BlockSpec((GB, H), lambda i: (i, 0)),
            full(k1w), full(row(k1b)), full(k2w), full(row(k2b)),
            full(sw), full(row(sb)), full(row(ng)), full(row(nb)),
        ],
        out_specs=pl.BlockSpec((GB, H), lambda i: (i, 0)),
        out_shape=jax.ShapeDtypeStruct((N, H), jnp.float32),
    )(nbrh, nbrc, coord_e, hq, k1w, row(k1b), k2w, row(k2b),
      sw, row(sb), row(ng), row(nb))


def _head_body(h_ref, w1_ref, b1_ref, w2_ref, b2_ref, out_ref):
    t = (
        jnp.dot(h_ref[...], w1_ref[...], preferred_element_type=jnp.float32)
        + b1_ref[...]
    )
    g = _gelu_exact(t)
    s = (
        jnp.dot(g, w2_ref[...], preferred_element_type=jnp.float32)
        + b2_ref[...]
    )
    out_ref[...] = jax.nn.sigmoid(s)


def _head(h, w1, b1, w2, b2):
    row = lambda a: a.reshape(1, -1)
    full = lambda a: pl.BlockSpec(a.shape, lambda i: (0, 0))
    return pl.pallas_call(
        _head_body,
        grid=(N // LB,),
        in_specs=[
            pl.BlockSpec((LB, H), lambda i: (i, 0)),
            full(w1), full(row(b1)), full(w2), full(row(b2)),
        ],
        out_specs=pl.BlockSpec((LB, 1), lambda i: (i, 0)),
        out_shape=jax.ShapeDtypeStruct((N, 1), jnp.float32),
    )(h, w1, row(b1), w2, row(b2))


def _rank_body(si_ref, st_ref, out_ref):
    pid = pl.program_id(0)
    si = si_ref[:, 0:1]
    sj = st_ref[0:1, :]
    iidx = pid * RB + lax.broadcasted_iota(jnp.int32, (RB, 1), 0)
    jidx = lax.broadcasted_iota(jnp.int32, (1, NP), 1)
    lt = (sj < si).astype(jnp.int32)
    eq = ((sj == si) & (jidx < iidx)).astype(jnp.int32)
    out_ref[...] = jnp.sum(lt + eq, axis=1, keepdims=True)


def _rank(scores, scores_t):
    return pl.pallas_call(
        _rank_body,
        grid=(N // RB,),
        in_specs=[
            pl.BlockSpec((RB, 1), lambda i: (i, 0)),
            pl.BlockSpec((8, NP), lambda i: (0, 0)),
        ],
        out_specs=pl.BlockSpec((RB, 1), lambda i: (i, 0)),
        out_shape=jax.ShapeDtypeStruct((N, 1), jnp.int32),
    )(scores, scores_t)


NSC = NP // 128


def _sc_scatter_order(rank2d, vals2d):
    mesh = plsc.VectorSubcoreMesh(core_axis_name="c", subcore_axis_name="s")

    @functools.partial(
        pl.kernel,
        out_type=jax.ShapeDtypeStruct((NP, 16), jnp.int32),
        mesh=mesh,
        compiler_params=pltpu.CompilerParams(use_tc_tiling_on_sc=False),
        scratch_types=[
            pltpu.VMEM((128,), jnp.int32),
            pltpu.VMEM((128, 16), jnp.int32),
            pltpu.SemaphoreType.DMA,
        ],
    )
    def sk(rank_hbm, vals_hbm, out_hbm, idx_v, rows_v, sem):
        wid = lax.axis_index("s") * 2 + lax.axis_index("c")
        for t in range((NSC + NW - 1) // NW):
            c = wid + NW * t

            @pl.when(c < NSC)
            def _():
                pltpu.sync_copy(rank_hbm.at[c], idx_v)
                pltpu.sync_copy(vals_hbm.at[pl.ds(c * 128, 128)], rows_v)
                pltpu.async_copy(rows_v, out_hbm.at[idx_v], sem).wait()

    return sk(rank2d, vals2d)


def kernel(coord, feat, offset, batch, lift_W, lift_b,
           g1k1_W, g1k1_b, g1k2_W, g1k2_b, g1s_W, g1s_b,
           g2k1_W, g2k1_b, g2k2_W, g2k2_b, g2s_W, g2s_b,
           n1_g, n1_b, n2_g, n2_b, sh1_W, sh1_b, sh2_W, sh2_b):
    f32 = jnp.float32

    inp = jnp.concatenate([coord, feat], axis=1)
    h = _lift(inp, lift_W, lift_b)

    coord_q = jnp.pad(coord, ((0, 0), (0, 5)))
    ct = jnp.pad(coord.T, ((0, 5), (0, NP - N)), constant_values=1e5)
    idx = _knn(coord_q, ct)

    idx_pad = jnp.concatenate(
        [idx.reshape(-1), jnp.zeros((E_PAD - E,), jnp.int32)]
    )

    coord16 = jnp.pad(coord, ((0, 0), (0, 13)))
    nbrc = _sc_gather(idx_pad, coord16)[:E]
    coord_e = jnp.repeat(coord_q, K, axis=0)

    for (k1w, k1b, k2w, k2b, sw, sb, ng, nb) in (
        (g1k1_W, g1k1_b, g1k2_W, g1k2_b, g1s_W, g1s_b, n1_g, n1_b),
        (g2k1_W, g2k1_b, g2k2_W, g2k2_b, g2s_W, g2s_b, n2_g, n2_b),
    ):
        nbrh = _sc_gather(idx_pad, h)[:E]
        h = _gno(nbrh, nbrc, coord_e, h, k1w, k1b, k2w, k2b, sw, sb, ng, nb)

    scores = _head(h, sh1_W, sh1_b, sh2_W, sh2_b)

    st = jnp.pad(scores.T, ((0, 7), (0, NP - N)), constant_values=np.inf)
    rank = _rank(scores, st)[:, 0]

    rank_pad = jnp.concatenate(
        [rank, jnp.arange(N, NP, dtype=jnp.int32)]
    ).reshape(NSC, 128)
    vals2d = jnp.broadcast_to(
        jnp.arange(NP, dtype=jnp.int32)[:, None], (NP, 16)
    )
    order = _sc_scatter_order(rank_pad, vals2d)[:N, 0]

    return scores, order[None, :], rank[None, :]


# speedup vs baseline: 1.0602x; 1.0602x over previous
import functools

import numpy as np
import jax
import jax.numpy as jnp
from jax import lax
from jax.experimental import pallas as pl
from jax.experimental.pallas import tpu as pltpu
from jax.experimental.pallas import tpu_sc as plsc

N = 10000
K = 16
H = 64
NP = 10240
QB = 200
GB = 200
LB = 1000
RB = 1000
E = N * K
E_PAD = 163840
NW = 32
PER_W = E_PAD // NW
SUB = 512

_SQRT_HALF = np.sqrt(0.5).astype(np.float32)
_BIG = float(np.inf)

_P9 = [2.326819970068386e-2, -1.387039388740657e-1, 3.687424674597105e-1,
       -5.824733027278666e-1, 6.210004621745983e-1, -4.944515323274145e-1,
       3.404879937665872e-1, -2.741127028184656e-1, 5.638259427386472e-1]
_R8 = [-1.047766399936249e+1, 1.297719955372516e+1, -7.495518717768503e+0,
       2.921019019210786e+0, -1.015265279202700e+0, 4.218463358204948e-1,
       -2.820767439740514e-1, 5.641895067754075e-1]
_T7 = [7.853861353153693e-5, -8.010193625184903e-4, 5.188327685732524e-3,
       -2.685381193529856e-2, 1.128358514861418e-1, -3.761262582423300e-1,
       1.128379165726710e+0]


def _poly(y, coeffs):
    p = jnp.zeros_like(y)
    for c in coeffs:
        p = p * y + np.float32(c)
    return p


def _erfc(w):
    ax = jnp.abs(w)
    xsq = w * w
    z = jnp.exp(-xsq)
    q = 1.0 / ax
    y = 1.0 / xsq
    p = jnp.where(ax < 2.0, _poly(y, _P9), _poly(y, _R8))
    yv = (z * q) * p
    yc = jnp.where(w < 0.0, 2.0 - yv, yv)
    sm = 1.0 - w * _poly(xsq, _T7)
    return jnp.where(ax < 1.0, sm, yc)


def _gelu_exact(x):
    return 0.5 * x * _erfc((-x) * _SQRT_HALF)


def _lane_sum64(a):
    acc = a[:, 0:8]
    for t in range(1, 8):
        acc = acc + a[:, 8 * t : 8 * t + 8]
    acc = acc[:, 0:4] + acc[:, 4:8]
    acc = acc[:, 0:2] + acc[:, 2:4]
    return acc[:, 0:1] + acc[:, 1:2]


def _mean_axis1_16(t3):
    acc = t3[:, 0, :]
    for j in range(1, K):
        acc = acc + t3[:, j, :]
    return acc / 16.0


def _lift_body(inp_ref, w_ref, b_ref, out_ref):
    out_ref[...] = (
        jnp.dot(inp_ref[...], w_ref[...], preferred_element_type=jnp.float32)
        + b_ref[...]
    )


def _lift(inp, w, b):
    return pl.pallas_call(
        _lift_body,
        grid=(N // LB,),
        in_specs=[
            pl.BlockSpec((LB, inp.shape[1]), lambda i: (i, 0)),
            pl.BlockSpec(w.shape, lambda i: (0, 0)),
            pl.BlockSpec((1, w.shape[1]), lambda i: (0, 0)),
        ],
        out_specs=pl.BlockSpec((LB, w.shape[1]), lambda i: (i, 0)),
        out_shape=jax.ShapeDtypeStruct((N, w.shape[1]), jnp.float32),
    )(inp, w, b.reshape(1, -1))


def _knn_body(cq_ref, ct_ref, idx_ref):
    qx = cq_ref[:, 0:1]
    qy = cq_ref[:, 1:2]
    qz = cq_ref[:, 2:3]
    cx = ct_ref[0:1, :]
    cy = ct_ref[1:2, :]
    cz = ct_ref[2:3, :]
    dx = qx - cx
    dy = qy - cy
    dz = qz - cz
    d = (dx * dx + dy * dy) + dz * dz
    lane = lax.broadcasted_iota(jnp.int32, (QB, NP), 1)
    for k in range(K):
        pos = jnp.argmin(d, axis=1).astype(jnp.int32)
        idx_ref[:, k : k + 1] = pos[:, None]
        d = jnp.where(lane == pos[:, None], _BIG, d)


def _knn(coord_q, coord_t):
    return pl.pallas_call(
        _knn_body,
        grid=(N // QB,),
        in_specs=[
            pl.BlockSpec((QB, 8), lambda i: (i, 0)),
            pl.BlockSpec((8, NP), lambda i: (0, 0)),
        ],
        out_specs=pl.BlockSpec((QB, K), lambda i: (i, 0)),
        out_shape=jax.ShapeDtypeStruct((N, K), jnp.int32),
    )(coord_q, coord_t)


def _sc_gather(idx_pad, table):
    d = table.shape[1]
    mesh = plsc.VectorSubcoreMesh(core_axis_name="c", subcore_axis_name="s")

    n_chunks = PER_W // SUB

    @functools.partial(
        pl.kernel,
        out_type=jax.ShapeDtypeStruct((E_PAD, d), jnp.float32),
        mesh=mesh,
        compiler_params=pltpu.CompilerParams(use_tc_tiling_on_sc=False),
        scratch_types=[
            pltpu.VMEM((PER_W,), jnp.int32),
            pltpu.VMEM((2, SUB, d), jnp.float32),
            pltpu.SemaphoreType.DMA,
            pltpu.SemaphoreType.DMA,
        ],
    )
    def gk(idx_hbm, table_hbm, out_hbm, idx_v, rows_v, gsem, ssem):
        wid = lax.axis_index("s") * 2 + lax.axis_index("c")
        base_w = wid * PER_W
        pltpu.sync_copy(idx_hbm.at[pl.ds(base_w, PER_W)], idx_v)

        def fire(t, b):
            hs = []
            for q in range(SUB // 128):
                hs.append(
                    pltpu.async_copy(
                        table_hbm.at[idx_v.at[pl.ds(t * SUB + q * 128, 128)]],
                        rows_v.at[b].at[pl.ds(q * 128, 128)],
                        gsem,
                    )
                )
            return hs

        gh = {0: fire(0, 0)}
        sh = {}
        for t in range(n_chunks):
            b = t % 2
            if t + 1 < n_chunks:
                if t >= 1:
                    sh.pop(t - 1).wait()
                gh[t + 1] = fire(t + 1, 1 - b)
            for hnd in gh.pop(t):
                hnd.wait()
            sh[t] = pltpu.async_copy(
                rows_v.at[b], out_hbm.at[pl.ds(base_w + t * SUB, SUB)], ssem
            )
        sh.pop(n_chunks - 2).wait()
        sh.pop(n_chunks - 1).wait()

    return gk(idx_pad, table)


def _gno_body(nbrh_ref, nbrc_ref, ce_ref, hq_ref, k1w_ref, k1b_ref,
              k2w_ref, k2b_ref, sw_ref, sb_ref, ng_ref, nb_ref, out_ref):
    rel = nbrc_ref[:, 0:3] - ce_ref[:, 0:3]
    ki = jnp.concatenate([rel, nbrh_ref[...]], axis=1)
    t1 = (
        jnp.dot(ki, k1w_ref[...], preferred_element_type=jnp.float32)
        + k1b_ref[...]
    )
    g = _gelu_exact(t1)
    t2 = (
        jnp.dot(g, k2w_ref[...], preferred_element_type=jnp.float32)
        + k2b_ref[...]
    )
    agg = _mean_axis1_16(t2.reshape(GB, K, H))
    y = jnp.maximum(
        (agg + jnp.dot(hq_ref[...], sw_ref[...],
                       preferred_element_type=jnp.float32)) + sb_ref[...],
        0,
    )
    m = _lane_sum64(y) / 64.0
    c = y - m
    v = _lane_sum64(c * c) / 64.0
    out_ref[...] = (y - m) / jnp.sqrt(v + 1e-5) * ng_ref[...] + nb_ref[...]


def _gno(nbrh, nbrc, coord_e, hq, k1w, k1b, k2w, k2b, sw, sb, ng, nb):
    row = lambda a: a.reshape(1, -1)
    full = lambda a: pl.BlockSpec(a.shape, lambda i: (0, 0))
    return pl.pallas_call(
        _gno_body,
        grid=(N // GB,),
        in_specs=[
            pl.BlockSpec((GB * K, H), lambda i: (i, 0)),
            pl.BlockSpec((GB * K, 16), lambda i: (i, 0)),
            pl.BlockSpec((GB * K, 8), lambda i: (i, 0)),
            pl.BlockSpec((GB, H), lambda i: (i, 0)),
            full(k1w), full(row(k1b)), full(k2w), full(row(k2b)),
            full(sw), full(row(sb)), full(row(ng)), full(row(nb)),
        ],
        out_specs=pl.BlockSpec((GB, H), lambda i: (i, 0)),
        out_shape=jax.ShapeDtypeStruct((N, H), jnp.float32),
    )(nbrh, nbrc, coord_e, hq, k1w, row(k1b), k2w, row(k2b),
      sw, row(sb), row(ng), row(nb))


def _head_body(h_ref, w1_ref, b1_ref, w2_ref, b2_ref, out_ref):
    t = (
        jnp.dot(h_ref[...], w1_ref[...], preferred_element_type=jnp.float32)
        + b1_ref[...]
    )
    g = _gelu_exact(t)
    s = (
        jnp.dot(g, w2_ref[...], preferred_element_type=jnp.float32)
        + b2_ref[...]
    )
    out_ref[...] = jax.nn.sigmoid(s)


def _head(h, w1, b1, w2, b2):
    row = lambda a: a.reshape(1, -1)
    full = lambda a: pl.BlockSpec(a.shape, lambda i: (0, 0))
    return pl.pallas_call(
        _head_body,
        grid=(N // LB,),
        in_specs=[
            pl.BlockSpec((LB, H), lambda i: (i, 0)),
            full(w1), full(row(b1)), full(w2), full(row(b2)),
        ],
        out_specs=pl.BlockSpec((LB, 1), lambda i: (i, 0)),
        out_shape=jax.ShapeDtypeStruct((N, 1), jnp.float32),
    )(h, w1, row(b1), w2, row(b2))


def _rank_body(si_ref, st_ref, out_ref):
    pid = pl.program_id(0)
    si = si_ref[:, 0:1]
    sj = st_ref[0:1, :]
    iidx = pid * RB + lax.broadcasted_iota(jnp.int32, (RB, 1), 0)
    jidx = lax.broadcasted_iota(jnp.int32, (1, NP), 1)
    lt = (sj < si).astype(jnp.int32)
    eq = ((sj == si) & (jidx < iidx)).astype(jnp.int32)
    out_ref[...] = jnp.sum(lt + eq, axis=1, keepdims=True)


def _rank(scores, scores_t):
    return pl.pallas_call(
        _rank_body,
        grid=(N // RB,),
        in_specs=[
            pl.BlockSpec((RB, 1), lambda i: (i, 0)),
            pl.BlockSpec((8, NP), lambda i: (0, 0)),
        ],
        out_specs=pl.BlockSpec((RB, 1), lambda i: (i, 0)),
        out_shape=jax.ShapeDtypeStruct((N, 1), jnp.int32),
    )(scores, scores_t)


NSC = NP // 128


def _sc_scatter_order(rank2d, vals2d):
    mesh = plsc.VectorSubcoreMesh(core_axis_name="c", subcore_axis_name="s")

    @functools.partial(
        pl.kernel,
        out_type=jax.ShapeDtypeStruct((NP, 16), jnp.int32),
        mesh=mesh,
        compiler_params=pltpu.CompilerParams(use_tc_tiling_on_sc=False),
        scratch_types=[
            pltpu.VMEM((128,), jnp.int32),
            pltpu.VMEM((128, 16), jnp.int32),
            pltpu.SemaphoreType.DMA,
        ],
    )
    def sk(rank_hbm, vals_hbm, out_hbm, idx_v, rows_v, sem):
        wid = lax.axis_index("s") * 2 + lax.axis_index("c")
        for t in range((NSC + NW - 1) // NW):
            c = wid + NW * t

            @pl.when(c < NSC)
            def _():
                pltpu.sync_copy(rank_hbm.at[c], idx_v)
                pltpu.sync_copy(vals_hbm.at[pl.ds(c * 128, 128)], rows_v)
                pltpu.async_copy(rows_v, out_hbm.at[idx_v], sem).wait()

    return sk(rank2d, vals2d)


def kernel(coord, feat, offset, batch, lift_W, lift_b,
           g1k1_W, g1k1_b, g1k2_W, g1k2_b, g1s_W, g1s_b,
           g2k1_W, g2k1_b, g2k2_W, g2k2_b, g2s_W, g2s_b,
           n1_g, n1_b, n2_g, n2_b, sh1_W, sh1_b, sh2_W, sh2_b):
    f32 = jnp.float32

    inp = jnp.concatenate([coord, feat], axis=1)
    h = _lift(inp, lift_W, lift_b)

    coord_q = jnp.pad(coord, ((0, 0), (0, 5)))
    ct = jnp.pad(coord.T, ((0, 5), (0, NP - N)), constant_values=1e5)
    idx = _knn(coord_q, ct)

    idx_pad = jnp.concatenate(
        [idx.reshape(-1), jnp.zeros((E_PAD - E,), jnp.int32)]
    )

    coord16 = jnp.pad(coord, ((0, 0), (0, 13)))
    nbrc = _sc_gather(idx_pad, coord16)[:E]
    coord_e = jnp.repeat(coord_q, K, axis=0)

    for (k1w, k1b, k2w, k2b, sw, sb, ng, nb) in (
        (g1k1_W, g1k1_b, g1k2_W, g1k2_b, g1s_W, g1s_b, n1_g, n1_b),
        (g2k1_W, g2k1_b, g2k2_W, g2k2_b, g2s_W, g2s_b, n2_g, n2_b),
    ):
        nbrh = _sc_gather(idx_pad, h)[:E]
        h = _gno(nbrh, nbrc, coord_e, h, k1w, k1b, k2w, k2b, sw, sb, ng, nb)

    scores = _head(h, sh1_W, sh1_b, sh2_W, sh2_b)

    st = jnp.pad(scores.T, ((0, 7), (0, NP - N)), constant_values=np.inf)
    rank = _rank(scores, st)[:, 0]

    rank_pad = jnp.concatenate(
        [rank, jnp.arange(N, NP, dtype=jnp.int32)]
    ).reshape(NSC, 128)
    vals2d = jnp.broadcast_to(
        jnp.arange(NP, dtype=jnp.int32)[:, None], (NP, 16)
    )
    order = _sc_scatter_order(rank_pad, vals2d)[:N, 0]

    return scores, order[None, :], rank[None, :]
